# Initial kernel scaffold; baseline (speedup 1.0000x reference)
import jax
import jax.numpy as jnp
from jax.experimental import pallas as pl


def kernel(inputs, embedding):
    raise NotImplementedError("write your pallas kernel here")


# speedup vs baseline: 1.1906x; 1.1906x over previous
import functools

import jax
import jax.numpy as jnp
from jax.experimental import pallas as pl
from jax.experimental.pallas import tpu as pltpu

_SOM_H = 32
_SOM_W = 32
_NUM_EMB = _SOM_H * _SOM_W
_EMB_DIM = 256
_ALPHA = 6.0
_BETA = 1.0
_PIX = 32 * 32
_B = 2
_D = 8
_N_ROWS = _B * _D * _PIX


def _body(x_ref, e_ref, q_ref, enc_ref, acc_ref):
    g = pl.program_id(0)
    xT = x_ref[0, :, 0, 0, :]
    e = e_ref[...]

    e_bf = e.astype(jnp.bfloat16)
    x_bf = xT.astype(jnp.bfloat16)
    dot = jax.lax.dot_general(e_bf, x_bf, (((1,), (0,)), ((), ())),
                              preferred_element_type=jnp.float32)
    e2 = jnp.sum(e * e, axis=1, keepdims=True)
    x2 = jnp.sum(xT * xT, axis=0, keepdims=True)
    distT = (x2 + e2) - 2.0 * dot

    minv = jnp.min(distT, axis=0, keepdims=True)
    j_iota = jax.lax.broadcasted_iota(jnp.int32, (_NUM_EMB, _PIX), 0)
    idx_row = jnp.min(jnp.where(distT == minv, j_iota, _NUM_EMB),
                      axis=0, keepdims=True)

    onehotT = j_iota == idx_row
    qT = jax.lax.dot_general(e_bf, onehotT.astype(jnp.bfloat16),
                             (((0,), (0,)), ((), ())),
                             preferred_element_type=jnp.float32)
    q_ref[0, :, 0, 0, :] = qT

    r_iota = jax.lax.broadcasted_iota(jnp.int32, (_PIX, _NUM_EMB), 1)
    enc_ref[...] = (r_iota == idx_row.reshape(_PIX, 1)).astype(jnp.float32)

    col = idx_row % _SOM_W
    ind = (onehotT
           | ((j_iota == idx_row - _SOM_W) & (idx_row >= _SOM_W))
           | ((j_iota == idx_row + _SOM_W) & (idx_row < _NUM_EMB - _SOM_W))
           | ((j_iota == idx_row - 1) & (col != 0))
           | ((j_iota == idx_row + 1) & (col != _SOM_W - 1)))
    indf = ind.astype(jnp.float32)
    som_num = jnp.sum(distT * indf)
    som_den = jnp.sum(indf)
    commit = jnp.sum((qT - xT) ** 2)

    @pl.when(g == 0)
    def _():
        acc_ref[...] = jnp.zeros_like(acc_ref)

    row_i = jax.lax.broadcasted_iota(jnp.int32, (8, 128), 0)
    lane_i = jax.lax.broadcasted_iota(jnp.int32, (8, 128), 1)
    first = lane_i == 0
    acc_ref[...] += (jnp.where((row_i == 0) & first, commit, 0.0)
                     + jnp.where((row_i == 1) & first, som_num, 0.0)
                     + jnp.where((row_i == 2) & first, som_den, 0.0))


@jax.jit
def kernel(inputs, embedding):
    x4 = inputs.reshape(_B, _EMB_DIM, _D, 1, _PIX)
    grid = _B * _D
    q4, enc, acc = pl.pallas_call(
        _body,
        grid=(grid,),
        in_specs=[
            pl.BlockSpec((1, _EMB_DIM, 1, 1, _PIX),
                         lambda g: (g // _D, 0, g % _D, 0, 0)),
            pl.BlockSpec((_NUM_EMB, _EMB_DIM), lambda g: (0, 0)),
        ],
        out_specs=[
            pl.BlockSpec((1, _EMB_DIM, 1, 1, _PIX),
                         lambda g: (g // _D, 0, g % _D, 0, 0)),
            pl.BlockSpec((_PIX, _NUM_EMB), lambda g: (g, 0)),
            pl.BlockSpec((8, 128), lambda g: (0, 0)),
        ],
        out_shape=[
            jax.ShapeDtypeStruct((_B, _EMB_DIM, _D, 1, _PIX), jnp.float32),
            jax.ShapeDtypeStruct((_N_ROWS, _NUM_EMB), jnp.float32),
            jax.ShapeDtypeStruct((8, 128), jnp.float32),
        ],
    )(x4, embedding)
    commit_sse = acc[0, 0]
    som_num = acc[1, 0]
    som_den = acc[2, 0]
    loss = _ALPHA * commit_sse / (_N_ROWS * _EMB_DIM) + _BETA * som_num / som_den
    out_q = q4.reshape(_B, _EMB_DIM, _D, _SOM_H, _SOM_W)
    return (loss, out_q, enc)


# speedup vs baseline: 1.3162x; 1.1055x over previous
import functools

import numpy as np

import jax
import jax.numpy as jnp
from jax.experimental import pallas as pl
from jax.experimental.pallas import tpu as pltpu

_SOM_H = 32
_SOM_W = 32
_NUM_EMB = _SOM_H * _SOM_W
_EMB_DIM = 256
_ALPHA = 6.0
_BETA = 1.0
_PIX = 32 * 32
_B = 2
_D = 8
_N_ROWS = _B * _D * _PIX


def _build_adj() -> np.ndarray:
    k = _NUM_EMB
    a = np.zeros((k, k), dtype=np.float32)
    for idx in range(k):
        i, j = divmod(idx, _SOM_W)
        a[idx, idx] = 1.0
        if i - 1 >= 0:
            a[idx, (i - 1) * _SOM_W + j] = 1.0
        if i + 1 < _SOM_H:
            a[idx, (i + 1) * _SOM_W + j] = 1.0
        if j - 1 >= 0:
            a[idx, i * _SOM_W + (j - 1)] = 1.0
        if j + 1 < _SOM_W:
            a[idx, i * _SOM_W + (j + 1)] = 1.0
    return a


_ADJ_NP = _build_adj()


def _body(x_ref, e_ref, adj_ref, q_ref, enc_ref, acc_ref):
    g = pl.program_id(0)
    xT = x_ref[0, :, 0, 0, :]
    e = e_ref[...]

    e_bf = e.astype(jnp.bfloat16)
    x_bf = xT.astype(jnp.bfloat16)
    dot = jax.lax.dot_general(e_bf, x_bf, (((1,), (0,)), ((), ())),
                              preferred_element_type=jnp.float32)
    e2 = jnp.sum(e * e, axis=1, keepdims=True)
    x2 = jnp.sum(xT * xT, axis=0, keepdims=True)
    distT = (x2 + e2) - 2.0 * dot

    minv = jnp.min(distT, axis=0, keepdims=True)
    j_iota = jax.lax.broadcasted_iota(jnp.int32, (_NUM_EMB, _PIX), 0)
    idx_row = jnp.min(jnp.where(distT == minv, j_iota, _NUM_EMB),
                      axis=0, keepdims=True)

    onehotT = (j_iota == idx_row).astype(jnp.bfloat16)
    qT = jax.lax.dot_general(e_bf, onehotT,
                             (((0,), (0,)), ((), ())),
                             preferred_element_type=jnp.float32)
    q_ref[0, :, 0, 0, :] = qT

    r_iota = jax.lax.broadcasted_iota(jnp.int32, (_PIX, _NUM_EMB), 1)
    enc_ref[...] = jnp.where(r_iota == idx_row.reshape(_PIX, 1), 1.0, 0.0)

    indf = jax.lax.dot_general(adj_ref[...], onehotT,
                               (((1,), (0,)), ((), ())),
                               preferred_element_type=jnp.float32)
    som_num = jnp.sum(distT * indf)
    som_den = jnp.sum(indf)
    commit = jnp.sum((qT - xT) ** 2)

    @pl.when(g == 0)
    def _():
        acc_ref[...] = jnp.zeros_like(acc_ref)

    row_i = jax.lax.broadcasted_iota(jnp.int32, (8, 128), 0)
    lane_i = jax.lax.broadcasted_iota(jnp.int32, (8, 128), 1)
    first = lane_i == 0
    acc_ref[...] += (jnp.where((row_i == 0) & first, commit, 0.0)
                     + jnp.where((row_i == 1) & first, som_num, 0.0)
                     + jnp.where((row_i == 2) & first, som_den, 0.0))


@jax.jit
def kernel(inputs, embedding):
    x4 = inputs.reshape(_B, _EMB_DIM, _D, 1, _PIX)
    adj = jnp.asarray(_ADJ_NP, dtype=jnp.bfloat16)
    grid = _B * _D
    q4, enc, acc = pl.pallas_call(
        _body,
        grid=(grid,),
        in_specs=[
            pl.BlockSpec((1, _EMB_DIM, 1, 1, _PIX),
                         lambda g: (g // _D, 0, g % _D, 0, 0)),
            pl.BlockSpec((_NUM_EMB, _EMB_DIM), lambda g: (0, 0)),
            pl.BlockSpec((_NUM_EMB, _NUM_EMB), lambda g: (0, 0)),
        ],
        out_specs=[
            pl.BlockSpec((1, _EMB_DIM, 1, 1, _PIX),
                         lambda g: (g // _D, 0, g % _D, 0, 0)),
            pl.BlockSpec((_PIX, _NUM_EMB), lambda g: (g, 0)),
            pl.BlockSpec((8, 128), lambda g: (0, 0)),
        ],
        out_shape=[
            jax.ShapeDtypeStruct((_B, _EMB_DIM, _D, 1, _PIX), jnp.float32),
            jax.ShapeDtypeStruct((_N_ROWS, _NUM_EMB), jnp.float32),
            jax.ShapeDtypeStruct((8, 128), jnp.float32),
        ],
    )(x4, embedding, adj)
    commit_sse = acc[0, 0]
    som_num = acc[1, 0]
    som_den = acc[2, 0]
    loss = _ALPHA * commit_sse / (_N_ROWS * _EMB_DIM) + _BETA * som_num / som_den
    out_q = q4.reshape(_B, _EMB_DIM, _D, _SOM_H, _SOM_W)
    return (loss, out_q, enc)


# speedup vs baseline: 2.5375x; 1.9279x over previous
import functools

import numpy as np

import jax
import jax.numpy as jnp
from jax.experimental import pallas as pl
from jax.experimental.pallas import tpu as pltpu

_SOM_H = 32
_SOM_W = 32
_NUM_EMB = _SOM_H * _SOM_W
_EMB_DIM = 256
_ALPHA = 6.0
_BETA = 1.0
_B = 2
_D = 8
_N_ROWS = _B * _D * _SOM_H * _SOM_W
_BLK = 2048
_GRID = _N_ROWS // _BLK


def _build_adj() -> np.ndarray:
    k = _NUM_EMB
    a = np.zeros((k, k), dtype=np.float32)
    for idx in range(k):
        i, j = divmod(idx, _SOM_W)
        a[idx, idx] = 1.0
        if i - 1 >= 0:
            a[idx, (i - 1) * _SOM_W + j] = 1.0
        if i + 1 < _SOM_H:
            a[idx, (i + 1) * _SOM_W + j] = 1.0
        if j - 1 >= 0:
            a[idx, i * _SOM_W + (j - 1)] = 1.0
        if j + 1 < _SOM_W:
            a[idx, i * _SOM_W + (j + 1)] = 1.0
    return a


_ADJ_NP = _build_adj()


def _body(x_ref, e_ref, adj_ref, q_ref, enc_ref, acc_ref):
    g = pl.program_id(0)
    x = x_ref[...]
    e = e_ref[...]

    e_bf = e.astype(jnp.bfloat16)
    x_bf = x.astype(jnp.bfloat16)
    dot = jax.lax.dot_general(x_bf, e_bf, (((1,), (1,)), ((), ())),
                              preferred_element_type=jnp.float32)
    x2 = jnp.sum(x * x, axis=1, keepdims=True)
    e2 = jnp.sum(e * e, axis=1).reshape(1, _NUM_EMB)
    dist = (x2 + e2) - 2.0 * dot

    minv = jnp.min(dist, axis=1, keepdims=True)
    j_iota = jax.lax.broadcasted_iota(jnp.int32, (_BLK, _NUM_EMB), 1)
    idx = jnp.min(jnp.where(dist == minv, j_iota, _NUM_EMB),
                  axis=1, keepdims=True)

    onehot = (j_iota == idx).astype(jnp.bfloat16)
    enc_ref[...] = onehot.astype(jnp.float32)

    q = jax.lax.dot_general(onehot, e_bf, (((1,), (0,)), ((), ())),
                            preferred_element_type=jnp.float32)
    q_ref[...] = q

    indf = jax.lax.dot_general(onehot, adj_ref[...], (((1,), (0,)), ((), ())),
                               preferred_element_type=jnp.float32)
    som_num = jnp.sum(dist * indf)
    som_den = jnp.sum(indf)
    commit = jnp.sum((q - x) ** 2)

    @pl.when(g == 0)
    def _():
        acc_ref[...] = jnp.zeros_like(acc_ref)

    row_i = jax.lax.broadcasted_iota(jnp.int32, (8, 128), 0)
    lane_i = jax.lax.broadcasted_iota(jnp.int32, (8, 128), 1)
    first = lane_i == 0
    acc_ref[...] += (jnp.where((row_i == 0) & first, commit, 0.0)
                     + jnp.where((row_i == 1) & first, som_num, 0.0)
                     + jnp.where((row_i == 2) & first, som_den, 0.0))


@jax.jit
def kernel(inputs, embedding):
    flat = jnp.transpose(inputs, (0, 2, 3, 4, 1)).reshape(_N_ROWS, _EMB_DIM)
    adj = jnp.asarray(_ADJ_NP, dtype=jnp.bfloat16)
    q, enc, acc = pl.pallas_call(
        _body,
        grid=(_GRID,),
        in_specs=[
            pl.BlockSpec((_BLK, _EMB_DIM), lambda g: (g, 0)),
            pl.BlockSpec((_NUM_EMB, _EMB_DIM), lambda g: (0, 0)),
            pl.BlockSpec((_NUM_EMB, _NUM_EMB), lambda g: (0, 0)),
        ],
        out_specs=[
            pl.BlockSpec((_BLK, _EMB_DIM), lambda g: (g, 0)),
            pl.BlockSpec((_BLK, _NUM_EMB), lambda g: (g, 0)),
            pl.BlockSpec((8, 128), lambda g: (0, 0)),
        ],
        out_shape=[
            jax.ShapeDtypeStruct((_N_ROWS, _EMB_DIM), jnp.float32),
            jax.ShapeDtypeStruct((_N_ROWS, _NUM_EMB), jnp.float32),
            jax.ShapeDtypeStruct((8, 128), jnp.float32),
        ],
    )(flat, embedding, adj)
    commit_sse = acc[0, 0]
    som_num = acc[1, 0]
    som_den = acc[2, 0]
    loss = _ALPHA * commit_sse / (_N_ROWS * _EMB_DIM) + _BETA * som_num / som_den
    out_q = jnp.transpose(q.reshape(_B, _D, _SOM_H, _SOM_W, _EMB_DIM),
                          (0, 4, 1, 2, 3))
    return (loss, out_q, enc)


# speedup vs baseline: 3.7032x; 1.4594x over previous
import functools

import numpy as np

import jax
import jax.numpy as jnp
from jax.experimental import pallas as pl
from jax.experimental.pallas import tpu as pltpu

_SOM_H = 32
_SOM_W = 32
_NUM_EMB = _SOM_H * _SOM_W
_EMB_DIM = 256
_ALPHA = 6.0
_BETA = 1.0
_B = 2
_D = 8
_N_ROWS = _B * _D * _SOM_H * _SOM_W
_BLK = 2048
_GRID = _N_ROWS // _BLK


def _build_adj() -> np.ndarray:
    k = _NUM_EMB
    a = np.zeros((k, k), dtype=np.float32)
    for idx in range(k):
        i, j = divmod(idx, _SOM_W)
        a[idx, idx] = 1.0
        if i - 1 >= 0:
            a[idx, (i - 1) * _SOM_W + j] = 1.0
        if i + 1 < _SOM_H:
            a[idx, (i + 1) * _SOM_W + j] = 1.0
        if j - 1 >= 0:
            a[idx, i * _SOM_W + (j - 1)] = 1.0
        if j + 1 < _SOM_W:
            a[idx, i * _SOM_W + (j + 1)] = 1.0
    return a


_ADJ_NP = _build_adj()


def _body(x_ref, e_ref, adj_ref, q_ref, enc_ref, acc_ref):
    g = pl.program_id(0)
    x = x_ref[...]
    e = e_ref[...]

    e_bf = e.astype(jnp.bfloat16)
    x_bf = x.astype(jnp.bfloat16)
    dot = jax.lax.dot_general(x_bf, e_bf, (((1,), (1,)), ((), ())),
                              preferred_element_type=jnp.float32)
    x2 = jnp.sum(x * x, axis=1, keepdims=True)
    e2 = jnp.sum(e * e, axis=1).reshape(1, _NUM_EMB)
    dist = (x2 + e2) - 2.0 * dot

    minv = jnp.min(dist, axis=1, keepdims=True)
    j_iota = jax.lax.broadcasted_iota(jnp.int32, (_BLK, _NUM_EMB), 1)
    idx = jnp.min(jnp.where(dist == minv, j_iota, _NUM_EMB),
                  axis=1, keepdims=True)

    onehot = (j_iota == idx).astype(jnp.bfloat16)
    enc_ref[...] = onehot.astype(jnp.float32)

    q = jax.lax.dot_general(onehot, e_bf, (((1,), (0,)), ((), ())),
                            preferred_element_type=jnp.float32)
    q_ref[...] = q

    som_num = jnp.float32(0.0)
    som_den = jnp.float32(0.0)
    for jb in range(_NUM_EMB // 128):
        j0 = jb * 128
        k0 = max(0, j0 - 128)
        k1 = min(_NUM_EMB, j0 + 256)
        ind_b = jax.lax.dot_general(onehot[:, k0:k1], adj_ref[k0:k1, j0:j0 + 128],
                                    (((1,), (0,)), ((), ())),
                                    preferred_element_type=jnp.float32)
        som_num += jnp.sum(dist[:, j0:j0 + 128] * ind_b)
        som_den += jnp.sum(ind_b)
    commit = jnp.sum((q - x) ** 2)

    @pl.when(g == 0)
    def _():
        acc_ref[...] = jnp.zeros_like(acc_ref)

    row_i = jax.lax.broadcasted_iota(jnp.int32, (8, 128), 0)
    lane_i = jax.lax.broadcasted_iota(jnp.int32, (8, 128), 1)
    first = lane_i == 0
    acc_ref[...] += (jnp.where((row_i == 0) & first, commit, 0.0)
                     + jnp.where((row_i == 1) & first, som_num, 0.0)
                     + jnp.where((row_i == 2) & first, som_den, 0.0))


@jax.jit
def kernel(inputs, embedding):
    flat = jnp.transpose(inputs, (0, 2, 3, 4, 1)).reshape(_N_ROWS, _EMB_DIM)
    adj = jnp.asarray(_ADJ_NP, dtype=jnp.bfloat16)
    q, enc, acc = pl.pallas_call(
        _body,
        grid=(_GRID,),
        in_specs=[
            pl.BlockSpec((_BLK, _EMB_DIM), lambda g: (g, 0)),
            pl.BlockSpec((_NUM_EMB, _EMB_DIM), lambda g: (0, 0)),
            pl.BlockSpec((_NUM_EMB, _NUM_EMB), lambda g: (0, 0)),
        ],
        out_specs=[
            pl.BlockSpec((_BLK, _EMB_DIM), lambda g: (g, 0)),
            pl.BlockSpec((_BLK, _NUM_EMB), lambda g: (g, 0)),
            pl.BlockSpec((8, 128), lambda g: (0, 0)),
        ],
        out_shape=[
            jax.ShapeDtypeStruct((_N_ROWS, _EMB_DIM), jnp.float32),
            jax.ShapeDtypeStruct((_N_ROWS, _NUM_EMB), jnp.float32),
            jax.ShapeDtypeStruct((8, 128), jnp.float32),
        ],
    )(flat, embedding, adj)
    commit_sse = acc[0, 0]
    som_num = acc[1, 0]
    som_den = acc[2, 0]
    loss = _ALPHA * commit_sse / (_N_ROWS * _EMB_DIM) + _BETA * som_num / som_den
    out_q = jnp.transpose(q.reshape(_B, _D, _SOM_H, _SOM_W, _EMB_DIM),
                          (0, 4, 1, 2, 3))
    return (loss, out_q, enc)
